# Initial kernel scaffold; baseline (speedup 1.0000x reference)
import jax
import jax.numpy as jnp
from jax.experimental import pallas as pl


def kernel(verts, feats, faces, edges, faces_to_edges):
    raise NotImplementedError("write your pallas kernel here")


# speedup vs baseline: 3.0253x; 3.0253x over previous
import functools

import jax
import jax.numpy as jnp
from jax import lax
from jax.experimental import pallas as pl
from jax.experimental.pallas import tpu as pltpu
from jax.experimental.pallas import tpu_sc as plsc

NC, NS = 2, 16
NW = NC * NS
CH = 240
NSUB = 2
SUB = CH // NSUB
CC = 400
CCV = 5000
CHF = 2000


def _cdiv(a, b):
    return (a + b - 1) // b


@functools.lru_cache(maxsize=None)
def _make_sc_kernel(V, D, E, F):
    n_mid = E // CH
    n_cp = V // CC
    n_vcp = (3 * V) // CCV
    n_fc = F // CHF
    D16 = D // 16
    NV = V + E
    mesh = plsc.VectorSubcoreMesh(core_axis_name="c", subcore_axis_name="s")

    segs = (
        (0, 0, 0, 0, False), (1, 2, 1, 0, True), (1, 1, 2, 0, True),
        (0, 1, 0, 1, False), (1, 0, 1, 1, True), (1, 2, 2, 1, True),
        (0, 2, 0, 2, False), (1, 1, 1, 2, True), (1, 0, 2, 2, True),
        (1, 0, 0, 3, True), (1, 1, 1, 3, True), (1, 2, 2, 3, True),
    )

    @functools.partial(
        pl.kernel,
        out_type=(
            jax.ShapeDtypeStruct((NV, D), jnp.float32),
            jax.ShapeDtypeStruct((3 * NV,), jnp.float32),
            jax.ShapeDtypeStruct((12 * F,), jnp.int32),
        ),
        mesh=mesh,
        scratch_types=[
            pltpu.VMEM((CH,), jnp.int32),
            pltpu.VMEM((CH,), jnp.int32),
            pltpu.VMEM((CH,), jnp.int32),
            pltpu.VMEM((CC, D), jnp.float32),
            pltpu.VMEM((CH, D), jnp.float32),
            pltpu.VMEM((CH,), jnp.float32),
            pltpu.VMEM((CH,), jnp.float32),
            pltpu.VMEM((CCV,), jnp.float32),
            pltpu.VMEM((CHF,), jnp.int32),
        ],
    )
    def kern(feats_hbm, verts_tf, edges_f, faces_tf, fte_tf,
             out_feats, out_verts_tf, out_faces_f,
             idx0_v, idx1_v, idxs_v, fbuf0, fbuf1, vb0, vb1, vcbuf, ibuf):
        wid = lax.axis_index("s") * NC + lax.axis_index("c")

        def mid_body(i, carry):
            c = wid + i * NW

            @pl.when(c < n_mid)
            def _():
                base = c * CH
                pltpu.sync_copy(edges_f.at[pl.ds(base, CH)], idx0_v)
                pltpu.sync_copy(edges_f.at[pl.ds(E + base, CH)], idx1_v)
                for j in range(NSUB):
                    sl = pl.ds(j * SUB, SUB)
                    pltpu.sync_copy(feats_hbm.at[idx0_v.at[sl]],
                                    fbuf0.at[sl])
                    pltpu.sync_copy(feats_hbm.at[idx1_v.at[sl]],
                                    fbuf1.at[sl])

                def frow(r, carry2):
                    for g in range(D16):
                        sl2 = pl.ds(g * 16, 16)
                        a = fbuf0[r, sl2]
                        b = fbuf1[r, sl2]
                        fbuf0[r, sl2] = (a + b) * 0.5
                    return carry2

                lax.fori_loop(0, CH, frow, 0)
                pltpu.sync_copy(fbuf0.at[pl.ds(0, CH)],
                                out_feats.at[pl.ds(V + base, CH)])

                for coord in range(3):
                    for ep, idx_v in ((0, idx0_v), (1, idx1_v)):
                        def shift(k, carry2, idx_v=idx_v, coord=coord):
                            sl2 = pl.ds(k * 16, 16)
                            idxs_v[sl2] = idx_v[sl2] + (coord * V)
                            return carry2

                        lax.fori_loop(0, CH // 16, shift, 0)
                        dst = vb0 if ep == 0 else vb1
                        for j in range(NSUB):
                            sl = pl.ds(j * SUB, SUB)
                            pltpu.sync_copy(verts_tf.at[idxs_v.at[sl]],
                                            dst.at[sl])

                    def vavg(k, carry2):
                        sl2 = pl.ds(k * 16, 16)
                        vb0[sl2] = (vb0[sl2] + vb1[sl2]) * 0.5
                        return carry2

                    lax.fori_loop(0, CH // 16, vavg, 0)
                    pltpu.sync_copy(
                        vb0, out_verts_tf.at[pl.ds(coord * NV + V + base, CH)])

            return carry

        lax.fori_loop(0, _cdiv(n_mid, NW), mid_body, 0)

        def cp_body(i, carry):
            c = wid + i * NW

            @pl.when(c < n_cp)
            def _():
                base = c * CC
                pltpu.sync_copy(feats_hbm.at[pl.ds(base, CC)], fbuf0)
                pltpu.sync_copy(fbuf0, out_feats.at[pl.ds(base, CC)])

            return carry

        lax.fori_loop(0, _cdiv(n_cp, NW), cp_body, 0)

        def vcp_body(i, carry):
            c = wid + i * NW

            @pl.when(c < n_vcp)
            def _():
                coord = c // (V // CCV)
                j = c - coord * (V // CCV)
                src_off = coord * V + j * CCV
                dst_off = coord * NV + j * CCV
                pltpu.sync_copy(verts_tf.at[pl.ds(src_off, CCV)], vcbuf)
                pltpu.sync_copy(vcbuf, out_verts_tf.at[pl.ds(dst_off, CCV)])

            return carry

        lax.fori_loop(0, _cdiv(n_vcp, NW), vcp_body, 0)

        for is_fte, srow, drow, reg, addnv in segs:
            src = fte_tf if is_fte else faces_tf

            def fc_body(i, carry, src=src, srow=srow, drow=drow, reg=reg,
                        addnv=addnv):
                j = wid + i * NW

                @pl.when(j < n_fc)
                def _():
                    off = j * CHF
                    pltpu.sync_copy(src.at[pl.ds(srow * F + off, CHF)], ibuf)
                    if addnv:
                        def addb(k, carry2):
                            sl = pl.ds(k * 16, 16)
                            ibuf[sl] = ibuf[sl] + V
                            return carry2

                        lax.fori_loop(0, CHF // 16, addb, 0)
                    pltpu.sync_copy(
                        ibuf,
                        out_faces_f.at[pl.ds(drow * 4 * F + reg * F + off,
                                             CHF)])

                return carry

            lax.fori_loop(0, _cdiv(n_fc, NW), fc_body, 0)

    return kern


def kernel(verts, feats, faces, edges, faces_to_edges):
    V, D = feats.shape
    E = edges.shape[0]
    F = faces.shape[0]
    verts_tf = verts.T.reshape(-1)
    edges_f = edges.T.reshape(-1)
    faces_tf = faces.T.reshape(-1)
    fte_tf = faces_to_edges.T.reshape(-1)
    kern = _make_sc_kernel(V, D, E, F)
    out_feats, out_verts_tf, out_faces_f = kern(feats, verts_tf, edges_f,
                                                faces_tf, fte_tf)
    new_verts = out_verts_tf.reshape(3, V + E).T
    new_faces = out_faces_f.reshape(3, 4 * F).T
    return new_verts, new_faces, out_feats


# speedup vs baseline: 5.3262x; 1.7605x over previous
import functools

import jax
import jax.numpy as jnp
from jax import lax
from jax.experimental import pallas as pl
from jax.experimental.pallas import tpu as pltpu
from jax.experimental.pallas import tpu_sc as plsc

NC, NS = 2, 16
NW = NC * NS
CH = 240
NSUB = 2
SUB = CH // NSUB
CC = 400
CCV = 5000
CHF = 2000


def _cdiv(a, b):
    return (a + b - 1) // b


@functools.lru_cache(maxsize=None)
def _make_sc_kernel(V, D, E, F):
    n_mid = E // CH
    n_cp = V // CC
    n_vcp = (3 * V) // CCV
    n_fc = F // CHF
    D16 = D // 16
    NV = V + E
    mesh = plsc.VectorSubcoreMesh(core_axis_name="c", subcore_axis_name="s")

    segs = (
        (0, 0, 0, 0, False), (1, 2, 1, 0, True), (1, 1, 2, 0, True),
        (0, 1, 0, 1, False), (1, 0, 1, 1, True), (1, 2, 2, 1, True),
        (0, 2, 0, 2, False), (1, 1, 1, 2, True), (1, 0, 2, 2, True),
        (1, 0, 0, 3, True), (1, 1, 1, 3, True), (1, 2, 2, 3, True),
    )

    @functools.partial(
        pl.kernel,
        out_type=(
            jax.ShapeDtypeStruct((NV, D), jnp.float32),
            jax.ShapeDtypeStruct((3 * NV,), jnp.float32),
            jax.ShapeDtypeStruct((12 * F,), jnp.int32),
        ),
        mesh=mesh,
        scratch_types=[
            pltpu.VMEM((CH,), jnp.int32),
            pltpu.VMEM((CH,), jnp.int32),
            pltpu.VMEM((CH,), jnp.int32),
            pltpu.VMEM((CH,), jnp.int32),
            pltpu.VMEM((CH,), jnp.int32),
            pltpu.VMEM((CH,), jnp.int32),
            pltpu.VMEM((CC, D), jnp.float32),
            pltpu.VMEM((CH, D), jnp.float32),
            pltpu.VMEM((CH,), jnp.float32),
            pltpu.VMEM((CH,), jnp.float32),
            pltpu.VMEM((CH,), jnp.float32),
            pltpu.VMEM((CH,), jnp.float32),
            pltpu.VMEM((CH,), jnp.float32),
            pltpu.VMEM((CH,), jnp.float32),
            pltpu.VMEM((CCV,), jnp.float32),
            pltpu.VMEM((CHF,), jnp.int32),
            pltpu.SemaphoreType.DMA,
        ],
    )
    def kern(feats_hbm, verts_tf, edges_f, faces_tf, fte_tf,
             out_feats, out_verts_tf, out_faces_f,
             idx0_v, idx1_v, is10, is11, is20, is21,
             fbuf0, fbuf1, vb00, vb01, vb02, vb10, vb11, vb12,
             vcbuf, ibuf, semg):
        vb0s = (vb00, vb01, vb02)
        vb1s = (vb10, vb11, vb12)
        wid = lax.axis_index("s") * NC + lax.axis_index("c")

        def mid_body(i, carry):
            c = wid + i * NW

            @pl.when(c < n_mid)
            def _():
                base = c * CH
                di0 = pltpu.async_copy(edges_f.at[pl.ds(base, CH)], idx0_v,
                                       semg)
                di1 = pltpu.async_copy(edges_f.at[pl.ds(E + base, CH)],
                                       idx1_v, semg)
                di0.wait()
                di1.wait()
                pend = []
                for j in range(NSUB):
                    sl = pl.ds(j * SUB, SUB)
                    pend.append(pltpu.async_copy(
                        feats_hbm.at[idx0_v.at[sl]], fbuf0.at[sl], semg))
                    pend.append(pltpu.async_copy(
                        feats_hbm.at[idx1_v.at[sl]], fbuf1.at[sl], semg))

                def shift(k, carry2):
                    sl2 = pl.ds(k * 16, 16)
                    a = idx0_v[sl2]
                    b = idx1_v[sl2]
                    is10[sl2] = a + V
                    is11[sl2] = b + V
                    is20[sl2] = a + 2 * V
                    is21[sl2] = b + 2 * V
                    return carry2

                lax.fori_loop(0, CH // 16, shift, 0)
                for coord, (ia, ib) in enumerate(
                        ((idx0_v, idx1_v), (is10, is11), (is20, is21))):
                    for j in range(NSUB):
                        sl = pl.ds(j * SUB, SUB)
                        pend.append(pltpu.async_copy(
                            verts_tf.at[ia.at[sl]], vb0s[coord].at[sl], semg))
                        pend.append(pltpu.async_copy(
                            verts_tf.at[ib.at[sl]], vb1s[coord].at[sl], semg))
                for d in pend:
                    d.wait()

                def frow(r, carry2):
                    for g in range(D16):
                        sl2 = pl.ds(g * 16, 16)
                        a = fbuf0[r, sl2]
                        b = fbuf1[r, sl2]
                        fbuf0[r, sl2] = (a + b) * 0.5
                    return carry2

                lax.fori_loop(0, CH, frow, 0)

                def vavg(k, carry2):
                    sl2 = pl.ds(k * 16, 16)
                    for coord in range(3):
                        vb0s[coord][sl2] = (vb0s[coord][sl2]
                                            + vb1s[coord][sl2]) * 0.5
                    return carry2

                lax.fori_loop(0, CH // 16, vavg, 0)
                pltpu.sync_copy(fbuf0.at[pl.ds(0, CH)],
                                out_feats.at[pl.ds(V + base, CH)])
                for coord in range(3):
                    pltpu.sync_copy(
                        vb0s[coord],
                        out_verts_tf.at[pl.ds(coord * NV + V + base, CH)])

            return carry

        lax.fori_loop(0, _cdiv(n_mid, NW), mid_body, 0)

        def cp_body(i, carry):
            c = wid + i * NW

            @pl.when(c < n_cp)
            def _():
                base = c * CC
                pltpu.sync_copy(feats_hbm.at[pl.ds(base, CC)], fbuf0)
                pltpu.sync_copy(fbuf0, out_feats.at[pl.ds(base, CC)])

            return carry

        lax.fori_loop(0, _cdiv(n_cp, NW), cp_body, 0)

        def vcp_body(i, carry):
            c = wid + i * NW

            @pl.when(c < n_vcp)
            def _():
                coord = c // (V // CCV)
                j = c - coord * (V // CCV)
                src_off = coord * V + j * CCV
                dst_off = coord * NV + j * CCV
                pltpu.sync_copy(verts_tf.at[pl.ds(src_off, CCV)], vcbuf)
                pltpu.sync_copy(vcbuf, out_verts_tf.at[pl.ds(dst_off, CCV)])

            return carry

        lax.fori_loop(0, _cdiv(n_vcp, NW), vcp_body, 0)

        for is_fte, srow, drow, reg, addnv in segs:
            src = fte_tf if is_fte else faces_tf

            def fc_body(i, carry, src=src, srow=srow, drow=drow, reg=reg,
                        addnv=addnv):
                j = wid + i * NW

                @pl.when(j < n_fc)
                def _():
                    off = j * CHF
                    pltpu.sync_copy(src.at[pl.ds(srow * F + off, CHF)], ibuf)
                    if addnv:
                        def addb(k, carry2):
                            sl = pl.ds(k * 16, 16)
                            ibuf[sl] = ibuf[sl] + V
                            return carry2

                        lax.fori_loop(0, CHF // 16, addb, 0)
                    pltpu.sync_copy(
                        ibuf,
                        out_faces_f.at[pl.ds(drow * 4 * F + reg * F + off,
                                             CHF)])

                return carry

            lax.fori_loop(0, _cdiv(n_fc, NW), fc_body, 0)

    return kern


def kernel(verts, feats, faces, edges, faces_to_edges):
    V, D = feats.shape
    E = edges.shape[0]
    F = faces.shape[0]
    verts_tf = verts.T.reshape(-1)
    edges_f = edges.T.reshape(-1)
    faces_tf = faces.T.reshape(-1)
    fte_tf = faces_to_edges.T.reshape(-1)
    kern = _make_sc_kernel(V, D, E, F)
    out_feats, out_verts_tf, out_faces_f = kern(feats, verts_tf, edges_f,
                                                faces_tf, fte_tf)
    new_verts = out_verts_tf.reshape(3, V + E).T
    new_faces = out_faces_f.reshape(3, 4 * F).T
    return new_verts, new_faces, out_feats


# speedup vs baseline: 6.0281x; 1.1318x over previous
import functools

import jax
import jax.numpy as jnp
from jax import lax
from jax.experimental import pallas as pl
from jax.experimental.pallas import tpu as pltpu
from jax.experimental.pallas import tpu_sc as plsc

NC, NS = 2, 16
NW = NC * NS
CH = 240
NSUB = 2
SUB = CH // NSUB
CC = 400
CCV = 5000
CHF = 2000


def _cdiv(a, b):
    return (a + b - 1) // b


@functools.lru_cache(maxsize=None)
def _make_sc_kernel(V, D, E, F):
    n_mid = E // CH
    n_cp = V // CC
    n_vcp = (3 * V) // CCV
    n_fc = F // CHF
    D16 = D // 16
    NV = V + E
    mesh = plsc.VectorSubcoreMesh(core_axis_name="c", subcore_axis_name="s")

    segs = (
        (0, 0, 0, 0, False), (1, 2, 1, 0, True), (1, 1, 2, 0, True),
        (0, 1, 0, 1, False), (1, 0, 1, 1, True), (1, 2, 2, 1, True),
        (0, 2, 0, 2, False), (1, 1, 1, 2, True), (1, 0, 2, 2, True),
        (1, 0, 0, 3, True), (1, 1, 1, 3, True), (1, 2, 2, 3, True),
    )

    @functools.partial(
        pl.kernel,
        out_type=(
            jax.ShapeDtypeStruct((NV, D), jnp.float32),
            jax.ShapeDtypeStruct((3 * NV,), jnp.float32),
            jax.ShapeDtypeStruct((12 * F,), jnp.int32),
        ),
        mesh=mesh,
        scratch_types=[
            pltpu.VMEM((CH,), jnp.int32),
            pltpu.VMEM((CH,), jnp.int32),
            pltpu.VMEM((CH,), jnp.int32),
            pltpu.VMEM((CH,), jnp.int32),
            pltpu.VMEM((CH,), jnp.int32),
            pltpu.VMEM((CH,), jnp.int32),
            pltpu.VMEM((CC, D), jnp.float32),
            pltpu.VMEM((CH, D), jnp.float32),
            pltpu.VMEM((CH,), jnp.float32),
            pltpu.VMEM((CH,), jnp.float32),
            pltpu.VMEM((CH,), jnp.float32),
            pltpu.VMEM((CH,), jnp.float32),
            pltpu.VMEM((CH,), jnp.float32),
            pltpu.VMEM((CH,), jnp.float32),
            pltpu.VMEM((CCV,), jnp.float32),
            pltpu.VMEM((CHF,), jnp.int32),
            pltpu.SemaphoreType.DMA,
            pltpu.SemaphoreType.DMA,
            pltpu.SemaphoreType.DMA,
        ],
    )
    def kern(feats_hbm, verts_tf, edges_f, faces_tf, fte_tf,
             out_feats, out_verts_tf, out_faces_f,
             idx0_v, idx1_v, is10, is11, is20, is21,
             fbuf0, fbuf1, vb00, vb01, vb02, vb10, vb11, vb12,
             vcbuf, ibuf, semg, semv, semw):
        vb0s = (vb00, vb01, vb02)
        vb1s = (vb10, vb11, vb12)
        wid = lax.axis_index("s") * NC + lax.axis_index("c")

        def mid_body(i, carry):
            c = wid + i * NW

            @pl.when(c < n_mid)
            def _():
                base = c * CH
                di0 = pltpu.async_copy(edges_f.at[pl.ds(base, CH)], idx0_v,
                                       semg)
                di1 = pltpu.async_copy(edges_f.at[pl.ds(E + base, CH)],
                                       idx1_v, semg)
                di0.wait()
                di1.wait()

                @pl.when(i > 0)
                def _():
                    pltpu.make_async_copy(
                        fbuf0.at[pl.ds(0, CH)],
                        out_feats.at[pl.ds(V + base, CH)], semw).wait()

                pendf = []
                for j in range(NSUB):
                    sl = pl.ds(j * SUB, SUB)
                    pendf.append(pltpu.async_copy(
                        feats_hbm.at[idx0_v.at[sl]], fbuf0.at[sl], semg))
                    pendf.append(pltpu.async_copy(
                        feats_hbm.at[idx1_v.at[sl]], fbuf1.at[sl], semg))

                def shift(k, carry2):
                    sl2 = pl.ds(k * 16, 16)
                    a = idx0_v[sl2]
                    b = idx1_v[sl2]
                    is10[sl2] = a + V
                    is11[sl2] = b + V
                    is20[sl2] = a + 2 * V
                    is21[sl2] = b + 2 * V
                    return carry2

                lax.fori_loop(0, CH // 16, shift, 0)
                pendv = []
                for coord, (ia, ib) in enumerate(
                        ((idx0_v, idx1_v), (is10, is11), (is20, is21))):
                    for j in range(NSUB):
                        sl = pl.ds(j * SUB, SUB)
                        pendv.append(pltpu.async_copy(
                            verts_tf.at[ia.at[sl]], vb0s[coord].at[sl], semv))
                        pendv.append(pltpu.async_copy(
                            verts_tf.at[ib.at[sl]], vb1s[coord].at[sl], semv))
                for d in pendf:
                    d.wait()

                def frow(r, carry2):
                    for g in range(D16):
                        sl2 = pl.ds(g * 16, 16)
                        a = fbuf0[r, sl2]
                        b = fbuf1[r, sl2]
                        fbuf0[r, sl2] = (a + b) * 0.5
                    return carry2

                lax.fori_loop(0, CH, frow, 0)
                pltpu.async_copy(fbuf0.at[pl.ds(0, CH)],
                                 out_feats.at[pl.ds(V + base, CH)], semw)

                for d in pendv:
                    d.wait()

                def vavg(k, carry2):
                    sl2 = pl.ds(k * 16, 16)
                    for coord in range(3):
                        vb0s[coord][sl2] = (vb0s[coord][sl2]
                                            + vb1s[coord][sl2]) * 0.5
                    return carry2

                lax.fori_loop(0, CH // 16, vavg, 0)
                for coord in range(3):
                    pltpu.sync_copy(
                        vb0s[coord],
                        out_verts_tf.at[pl.ds(coord * NV + V + base, CH)])

            return carry

        lax.fori_loop(0, _cdiv(n_mid, NW), mid_body, 0)
        pltpu.make_async_copy(fbuf0.at[pl.ds(0, CH)],
                              out_feats.at[pl.ds(0, CH)], semw).wait()

        def cp_body(i, carry):
            c = wid + i * NW

            @pl.when(c < n_cp)
            def _():
                base = c * CC
                pltpu.sync_copy(feats_hbm.at[pl.ds(base, CC)], fbuf0)
                pltpu.sync_copy(fbuf0, out_feats.at[pl.ds(base, CC)])

            return carry

        lax.fori_loop(0, _cdiv(n_cp, NW), cp_body, 0)

        def vcp_body(i, carry):
            c = wid + i * NW

            @pl.when(c < n_vcp)
            def _():
                coord = c // (V // CCV)
                j = c - coord * (V // CCV)
                src_off = coord * V + j * CCV
                dst_off = coord * NV + j * CCV
                pltpu.sync_copy(verts_tf.at[pl.ds(src_off, CCV)], vcbuf)
                pltpu.sync_copy(vcbuf, out_verts_tf.at[pl.ds(dst_off, CCV)])

            return carry

        lax.fori_loop(0, _cdiv(n_vcp, NW), vcp_body, 0)

        for is_fte, srow, drow, reg, addnv in segs:
            src = fte_tf if is_fte else faces_tf

            def fc_body(i, carry, src=src, srow=srow, drow=drow, reg=reg,
                        addnv=addnv):
                j = wid + i * NW

                @pl.when(j < n_fc)
                def _():
                    off = j * CHF
                    pltpu.sync_copy(src.at[pl.ds(srow * F + off, CHF)], ibuf)
                    if addnv:
                        def addb(k, carry2):
                            sl = pl.ds(k * 16, 16)
                            ibuf[sl] = ibuf[sl] + V
                            return carry2

                        lax.fori_loop(0, CHF // 16, addb, 0)
                    pltpu.sync_copy(
                        ibuf,
                        out_faces_f.at[pl.ds(drow * 4 * F + reg * F + off,
                                             CHF)])

                return carry

            lax.fori_loop(0, _cdiv(n_fc, NW), fc_body, 0)

    return kern


def kernel(verts, feats, faces, edges, faces_to_edges):
    V, D = feats.shape
    E = edges.shape[0]
    F = faces.shape[0]
    verts_tf = verts.T.reshape(-1)
    edges_f = edges.T.reshape(-1)
    faces_tf = faces.T.reshape(-1)
    fte_tf = faces_to_edges.T.reshape(-1)
    kern = _make_sc_kernel(V, D, E, F)
    out_feats, out_verts_tf, out_faces_f = kern(feats, verts_tf, edges_f,
                                                faces_tf, fte_tf)
    new_verts = out_verts_tf.reshape(3, V + E).T
    new_faces = out_faces_f.reshape(3, 4 * F).T
    return new_verts, new_faces, out_feats


# speedup vs baseline: 6.7606x; 1.1215x over previous
import functools

import jax
import jax.numpy as jnp
from jax import lax
from jax.experimental import pallas as pl
from jax.experimental.pallas import tpu as pltpu
from jax.experimental.pallas import tpu_sc as plsc

NC, NS = 2, 16
NW = NC * NS
CH = 240
NSUB = 2
SUB = CH // NSUB
CC = 200
CCV = 1000
CHF = 2000


def _cdiv(a, b):
    return (a + b - 1) // b


@functools.lru_cache(maxsize=None)
def _make_sc_kernel(V, D, E, F):
    n_mid = E // CH
    n_cp = V // CC
    n_vcp = (3 * V) // CCV
    n_fc = F // CHF
    D16 = D // 16
    NV = V + E
    mesh = plsc.VectorSubcoreMesh(core_axis_name="c", subcore_axis_name="s")

    segs = (
        (0, 0, 0, 0, False), (1, 2, 1, 0, True), (1, 1, 2, 0, True),
        (0, 1, 0, 1, False), (1, 0, 1, 1, True), (1, 2, 2, 1, True),
        (0, 2, 0, 2, False), (1, 1, 1, 2, True), (1, 0, 2, 2, True),
        (1, 0, 0, 3, True), (1, 1, 1, 3, True), (1, 2, 2, 3, True),
    )

    @functools.partial(
        pl.kernel,
        out_type=(
            jax.ShapeDtypeStruct((NV, D), jnp.float32),
            jax.ShapeDtypeStruct((3 * NV,), jnp.float32),
            jax.ShapeDtypeStruct((12 * F,), jnp.int32),
        ),
        mesh=mesh,
        scratch_types=[
            pltpu.VMEM((CH,), jnp.int32),
            pltpu.VMEM((CH,), jnp.int32),
            pltpu.VMEM((CH,), jnp.int32),
            pltpu.VMEM((CH,), jnp.int32),
            pltpu.VMEM((CH, D), jnp.float32),
            pltpu.VMEM((CH, D), jnp.float32),
            pltpu.VMEM((CH, D), jnp.float32),
            pltpu.VMEM((CH, D), jnp.float32),
            pltpu.VMEM((CH,), jnp.float32),
            pltpu.VMEM((CH,), jnp.float32),
            pltpu.VMEM((CH,), jnp.float32),
            pltpu.VMEM((CH,), jnp.float32),
            pltpu.VMEM((CH,), jnp.float32),
            pltpu.VMEM((CH,), jnp.float32),
            pltpu.VMEM((CH,), jnp.float32),
            pltpu.VMEM((CH,), jnp.float32),
            pltpu.VMEM((CH,), jnp.float32),
            pltpu.VMEM((CH,), jnp.float32),
            pltpu.VMEM((CH,), jnp.float32),
            pltpu.VMEM((CH,), jnp.float32),
            pltpu.VMEM((CCV,), jnp.float32),
            pltpu.VMEM((CHF,), jnp.int32),
            pltpu.SemaphoreType.DMA,
            pltpu.SemaphoreType.DMA,
            pltpu.SemaphoreType.DMA,
            pltpu.SemaphoreType.DMA,
            pltpu.SemaphoreType.DMA,
        ],
    )
    def kern(feats_hbm, vx, vy, vz, verts_tf, edges_f, faces_tf, fte_tf,
             out_feats, out_verts_tf, out_faces_f,
             i0_0, i1_0, i0_1, i1_1,
             fb0_0, fb1_0, fb0_1, fb1_1,
             p0x, p0y, p0z, q0x, q0y, q0z,
             p1x, p1y, p1z, q1x, q1y, q1z,
             vcbuf, ibuf, semg0, semg1, semv0, semv1, semw):
        wid = lax.axis_index("s") * NC + lax.axis_index("c")
        idxs = ((i0_0, i1_0), (i0_1, i1_1))
        fbs = ((fb0_0, fb1_0), (fb0_1, fb1_1))
        pbufs = ((p0x, p0y, p0z), (p1x, p1y, p1z))
        qbufs = ((q0x, q0y, q0z), (q1x, q1y, q1z))
        semg = (semg0, semg1)
        semv = (semv0, semv1)
        varrs = (vx, vy, vz)

        def mid_fire(t, S, drain):
            c = wid + t * NW

            @pl.when(c < n_mid)
            def _():
                base = c * CH
                i0, i1 = idxs[S]
                d0 = pltpu.async_copy(edges_f.at[pl.ds(base, CH)], i0,
                                      semg[S])
                d1 = pltpu.async_copy(edges_f.at[pl.ds(E + base, CH)], i1,
                                      semg[S])
                d0.wait()
                d1.wait()
                for j in range(NSUB):
                    sl = pl.ds(j * SUB, SUB)
                    pltpu.async_copy(feats_hbm.at[i1.at[sl]],
                                     fbs[S][1].at[sl], semg[S])
                    for coord in range(3):
                        pltpu.async_copy(varrs[coord].at[i0.at[sl]],
                                         pbufs[S][coord].at[sl], semv[S])
                        pltpu.async_copy(varrs[coord].at[i1.at[sl]],
                                         qbufs[S][coord].at[sl], semv[S])
                if drain:
                    pltpu.make_async_copy(
                        fbs[S][0].at[pl.ds(0, CH)],
                        out_feats.at[pl.ds(0, CH)], semw).wait()
                for j in range(NSUB):
                    sl = pl.ds(j * SUB, SUB)
                    pltpu.async_copy(feats_hbm.at[i0.at[sl]],
                                     fbs[S][0].at[sl], semg[S])

        def mid_finish(t, S):
            c = wid + t * NW

            @pl.when(c < n_mid)
            def _():
                base = c * CH
                i0, i1 = idxs[S]
                fb0, fb1 = fbs[S]
                for j in range(NSUB):
                    sl = pl.ds(j * SUB, SUB)
                    pltpu.make_async_copy(feats_hbm.at[i1.at[sl]],
                                          fb1.at[sl], semg[S]).wait()
                    pltpu.make_async_copy(feats_hbm.at[i0.at[sl]],
                                          fb0.at[sl], semg[S]).wait()

                def frow(r, carry2):
                    for g in range(D16):
                        sl2 = pl.ds(g * 16, 16)
                        a = fb0[r, sl2]
                        b = fb1[r, sl2]
                        fb0[r, sl2] = (a + b) * 0.5
                    return carry2

                lax.fori_loop(0, CH, frow, 0)
                pltpu.async_copy(fb0.at[pl.ds(0, CH)],
                                 out_feats.at[pl.ds(V + base, CH)], semw)

                for j in range(NSUB):
                    sl = pl.ds(j * SUB, SUB)
                    for coord in range(3):
                        pltpu.make_async_copy(
                            varrs[coord].at[i0.at[sl]],
                            pbufs[S][coord].at[sl], semv[S]).wait()
                        pltpu.make_async_copy(
                            varrs[coord].at[i1.at[sl]],
                            qbufs[S][coord].at[sl], semv[S]).wait()

                def vavg(k, carry2):
                    sl2 = pl.ds(k * 16, 16)
                    for coord in range(3):
                        pbufs[S][coord][sl2] = (pbufs[S][coord][sl2]
                                                + qbufs[S][coord][sl2]) * 0.5
                    return carry2

                lax.fori_loop(0, CH // 16, vavg, 0)
                for coord in range(3):
                    pltpu.sync_copy(
                        pbufs[S][coord],
                        out_verts_tf.at[pl.ds(coord * NV + V + base, CH)])

        T1 = _cdiv(n_mid, NW)
        mid_fire(0, 0, False)
        mid_fire(1, 1, False)

        def mid_pair(u, carry):
            t = 2 * u
            mid_finish(t, 0)
            mid_fire(t + 2, 0, True)
            mid_finish(t + 1, 1)
            mid_fire(t + 3, 1, True)
            return carry

        lax.fori_loop(0, _cdiv(T1, 2), mid_pair, 0)
        for _ in range(2):
            pltpu.make_async_copy(fb0_0.at[pl.ds(0, CH)],
                                  out_feats.at[pl.ds(0, CH)], semw).wait()

        def cp_fire(t, S, drain):
            c = wid + t * NW

            @pl.when(c < n_cp)
            def _():
                if drain:
                    pltpu.make_async_copy(
                        fbs[S][0].at[pl.ds(0, CC)],
                        out_feats.at[pl.ds(0, CC)], semw).wait()
                pltpu.async_copy(feats_hbm.at[pl.ds(c * CC, CC)],
                                 fbs[S][0].at[pl.ds(0, CC)], semg[S])

        def cp_finish(t, S):
            c = wid + t * NW

            @pl.when(c < n_cp)
            def _():
                pltpu.make_async_copy(feats_hbm.at[pl.ds(c * CC, CC)],
                                      fbs[S][0].at[pl.ds(0, CC)],
                                      semg[S]).wait()
                pltpu.async_copy(fbs[S][0].at[pl.ds(0, CC)],
                                 out_feats.at[pl.ds(c * CC, CC)], semw)

        T2 = _cdiv(n_cp, NW)
        cp_fire(0, 0, False)
        cp_fire(1, 1, False)

        def cp_pair(u, carry):
            t = 2 * u
            cp_finish(t, 0)
            cp_fire(t + 2, 0, True)
            cp_finish(t + 1, 1)
            cp_fire(t + 3, 1, True)
            return carry

        lax.fori_loop(0, _cdiv(T2, 2), cp_pair, 0)
        for _ in range(2):
            pltpu.make_async_copy(fb0_0.at[pl.ds(0, CC)],
                                  out_feats.at[pl.ds(0, CC)], semw).wait()

        def vcp_body(i, carry):
            c = wid + i * NW

            @pl.when(c < n_vcp)
            def _():
                @pl.when(i > 0)
                def _():
                    pltpu.make_async_copy(
                        vcbuf, out_verts_tf.at[pl.ds(0, CCV)], semw).wait()

                coord = c // (V // CCV)
                j = c - coord * (V // CCV)
                pltpu.sync_copy(verts_tf.at[pl.ds(coord * V + j * CCV, CCV)],
                                vcbuf)
                pltpu.async_copy(
                    vcbuf, out_verts_tf.at[pl.ds(coord * NV + j * CCV, CCV)],
                    semw)

            return carry

        lax.fori_loop(0, _cdiv(n_vcp, NW), vcp_body, 0)
        pltpu.make_async_copy(vcbuf, out_verts_tf.at[pl.ds(0, CCV)],
                              semw).wait()

        bodies = []
        for seg in segs:
            bodies.append((seg, 0))
            bodies.append((seg, 1))
        prev_fired = None
        for (is_fte, srow, drow, reg, addnv), ii in bodies:
            src = fte_tf if is_fte else faces_tf
            j = wid + ii * NW
            if prev_fired is not None:
                @pl.when(prev_fired < n_fc)
                def _():
                    pltpu.make_async_copy(
                        ibuf, out_faces_f.at[pl.ds(0, CHF)], semw).wait()

            @pl.when(j < n_fc)
            def _(src=src, srow=srow, drow=drow, reg=reg, addnv=addnv, j=j):
                off = j * CHF
                pltpu.sync_copy(src.at[pl.ds(srow * F + off, CHF)], ibuf)
                if addnv:
                    def addb(k, carry2):
                        sl = pl.ds(k * 16, 16)
                        ibuf[sl] = ibuf[sl] + V
                        return carry2

                    lax.fori_loop(0, CHF // 16, addb, 0)
                pltpu.async_copy(
                    ibuf,
                    out_faces_f.at[pl.ds(drow * 4 * F + reg * F + off, CHF)],
                    semw)

            prev_fired = j

        @pl.when(prev_fired < n_fc)
        def _():
            pltpu.make_async_copy(ibuf, out_faces_f.at[pl.ds(0, CHF)],
                                  semw).wait()

    return kern


def kernel(verts, feats, faces, edges, faces_to_edges):
    V, D = feats.shape
    E = edges.shape[0]
    F = faces.shape[0]
    verts_tf = verts.T.reshape(-1)
    edges_f = edges.T.reshape(-1)
    faces_tf = faces.T.reshape(-1)
    fte_tf = faces_to_edges.T.reshape(-1)
    kern = _make_sc_kernel(V, D, E, F)
    out_feats, out_verts_tf, out_faces_f = kern(
        feats, verts_tf[:V], verts_tf[V:2 * V], verts_tf[2 * V:], verts_tf,
        edges_f, faces_tf, fte_tf)
    new_verts = out_verts_tf.reshape(3, V + E).T
    new_faces = out_faces_f.reshape(3, 4 * F).T
    return new_verts, new_faces, out_feats


# speedup vs baseline: 7.0372x; 1.0409x over previous
import functools

import jax
import jax.numpy as jnp
from jax import lax
from jax.experimental import pallas as pl
from jax.experimental.pallas import tpu as pltpu
from jax.experimental.pallas import tpu_sc as plsc

NC, NS = 2, 16
NW = NC * NS
CH = 80
NSUB = 1
SUB = CH // NSUB
CC = 80
CCV = 1000
CHF = 2000


def _cdiv(a, b):
    return (a + b - 1) // b


@functools.lru_cache(maxsize=None)
def _make_sc_kernel(V, D, E, F):
    n_mid = E // CH
    n_cp = V // CC
    n_vcp = (3 * V) // CCV
    n_fc = F // CHF
    D16 = D // 16
    NV = V + E
    mesh = plsc.VectorSubcoreMesh(core_axis_name="c", subcore_axis_name="s")

    segs = (
        (0, 0, 0, 0, False), (1, 2, 1, 0, True), (1, 1, 2, 0, True),
        (0, 1, 0, 1, False), (1, 0, 1, 1, True), (1, 2, 2, 1, True),
        (0, 2, 0, 2, False), (1, 1, 1, 2, True), (1, 0, 2, 2, True),
        (1, 0, 0, 3, True), (1, 1, 1, 3, True), (1, 2, 2, 3, True),
    )

    @functools.partial(
        pl.kernel,
        out_type=(
            jax.ShapeDtypeStruct((NV, D), jnp.float32),
            jax.ShapeDtypeStruct((3 * NV,), jnp.float32),
            jax.ShapeDtypeStruct((12 * F,), jnp.int32),
        ),
        mesh=mesh,
        scratch_types=[
            pltpu.VMEM((CH,), jnp.int32),
            pltpu.VMEM((CH,), jnp.int32),
            pltpu.VMEM((CH,), jnp.int32),
            pltpu.VMEM((CH,), jnp.int32),
            pltpu.VMEM((CH, D), jnp.float32),
            pltpu.VMEM((CH, D), jnp.float32),
            pltpu.VMEM((CH, D), jnp.float32),
            pltpu.VMEM((CH, D), jnp.float32),
            pltpu.VMEM((CH,), jnp.float32),
            pltpu.VMEM((CH,), jnp.float32),
            pltpu.VMEM((CH,), jnp.float32),
            pltpu.VMEM((CH,), jnp.float32),
            pltpu.VMEM((CH,), jnp.float32),
            pltpu.VMEM((CH,), jnp.float32),
            pltpu.VMEM((CH,), jnp.float32),
            pltpu.VMEM((CH,), jnp.float32),
            pltpu.VMEM((CH,), jnp.float32),
            pltpu.VMEM((CH,), jnp.float32),
            pltpu.VMEM((CH,), jnp.float32),
            pltpu.VMEM((CH,), jnp.float32),
            pltpu.VMEM((CCV,), jnp.float32),
            pltpu.VMEM((CHF,), jnp.int32),
            pltpu.VMEM_SHARED((V,), jnp.float32),
            pltpu.VMEM_SHARED((V,), jnp.float32),
            pltpu.VMEM_SHARED((V,), jnp.float32),
            pltpu.SemaphoreType.DMA,
            pltpu.SemaphoreType.DMA,
            pltpu.SemaphoreType.DMA,
            pltpu.SemaphoreType.DMA,
            pltpu.SemaphoreType.DMA,
        ],
    )
    def kern(feats_hbm, vx, vy, vz, verts_tf, edges_f, faces_tf, fte_tf,
             out_feats, out_verts_tf, out_faces_f,
             i0_0, i1_0, i0_1, i1_1,
             fb0_0, fb1_0, fb0_1, fb1_1,
             p0x, p0y, p0z, q0x, q0y, q0z,
             p1x, p1y, p1z, q1x, q1y, q1z,
             vcbuf, ibuf, vsx, vsy, vsz, semg0, semg1, semv0, semv1,
             semw):
        wid = lax.axis_index("s") * NC + lax.axis_index("c")
        idxs = ((i0_0, i1_0), (i0_1, i1_1))
        fbs = ((fb0_0, fb1_0), (fb0_1, fb1_1))
        pbufs = ((p0x, p0y, p0z), (p1x, p1y, p1z))
        qbufs = ((q0x, q0y, q0z), (q1x, q1y, q1z))
        semg = (semg0, semg1)
        semv = (semv0, semv1)
        varrs = (vx, vy, vz)
        vsps = (vsx, vsy, vsz)

        sid = lax.axis_index("s")
        n_stg = V // CCV

        def stg_body(i, carry):
            t = sid + i * NS

            @pl.when(t < 3 * n_stg)
            def _():
                coord = t // n_stg
                j = t - coord * n_stg
                sl = pl.ds(j * CCV, CCV)
                for cc in range(3):
                    @pl.when(coord == cc)
                    def _(cc=cc):
                        pltpu.sync_copy(varrs[cc].at[sl], vcbuf)
                        pltpu.sync_copy(vcbuf, vsps[cc].at[sl])

            return carry

        lax.fori_loop(0, _cdiv(3 * (V // CCV), NS), stg_body, 0)
        plsc.subcore_barrier()

        def mid_fire(t, S, drain):
            c = wid + t * NW

            @pl.when(c < n_mid)
            def _():
                base = c * CH
                i0, i1 = idxs[S]
                d0 = pltpu.async_copy(edges_f.at[pl.ds(base, CH)], i0,
                                      semg[S])
                d1 = pltpu.async_copy(edges_f.at[pl.ds(E + base, CH)], i1,
                                      semg[S])
                d0.wait()
                d1.wait()
                for j in range(NSUB):
                    sl = pl.ds(j * SUB, SUB)
                    pltpu.async_copy(feats_hbm.at[i1.at[sl]],
                                     fbs[S][1].at[sl], semg[S])
                    for coord in range(3):
                        pltpu.async_copy(vsps[coord].at[i0.at[sl]],
                                         pbufs[S][coord].at[sl], semv[S])
                        pltpu.async_copy(vsps[coord].at[i1.at[sl]],
                                         qbufs[S][coord].at[sl], semv[S])
                if drain:
                    pltpu.make_async_copy(
                        fbs[S][0].at[pl.ds(0, CH)],
                        out_feats.at[pl.ds(0, CH)], semw).wait()
                for j in range(NSUB):
                    sl = pl.ds(j * SUB, SUB)
                    pltpu.async_copy(feats_hbm.at[i0.at[sl]],
                                     fbs[S][0].at[sl], semg[S])

        def mid_finish(t, S):
            c = wid + t * NW

            @pl.when(c < n_mid)
            def _():
                base = c * CH
                i0, i1 = idxs[S]
                fb0, fb1 = fbs[S]
                for j in range(NSUB):
                    sl = pl.ds(j * SUB, SUB)
                    pltpu.make_async_copy(feats_hbm.at[i1.at[sl]],
                                          fb1.at[sl], semg[S]).wait()
                    pltpu.make_async_copy(feats_hbm.at[i0.at[sl]],
                                          fb0.at[sl], semg[S]).wait()

                def frow(r, carry2):
                    for g in range(D16):
                        sl2 = pl.ds(g * 16, 16)
                        a = fb0[r, sl2]
                        b = fb1[r, sl2]
                        fb0[r, sl2] = (a + b) * 0.5
                    return carry2

                lax.fori_loop(0, CH, frow, 0)
                pltpu.async_copy(fb0.at[pl.ds(0, CH)],
                                 out_feats.at[pl.ds(V + base, CH)], semw)

                for j in range(NSUB):
                    sl = pl.ds(j * SUB, SUB)
                    for coord in range(3):
                        pltpu.make_async_copy(
                            vsps[coord].at[i0.at[sl]],
                            pbufs[S][coord].at[sl], semv[S]).wait()
                        pltpu.make_async_copy(
                            vsps[coord].at[i1.at[sl]],
                            qbufs[S][coord].at[sl], semv[S]).wait()

                def vavg(k, carry2):
                    sl2 = pl.ds(k * 16, 16)
                    for coord in range(3):
                        pbufs[S][coord][sl2] = (pbufs[S][coord][sl2]
                                                + qbufs[S][coord][sl2]) * 0.5
                    return carry2

                lax.fori_loop(0, CH // 16, vavg, 0)
                for coord in range(3):
                    pltpu.sync_copy(
                        pbufs[S][coord],
                        out_verts_tf.at[pl.ds(coord * NV + V + base, CH)])

        T1 = _cdiv(n_mid, NW)
        mid_fire(0, 0, False)
        mid_fire(1, 1, False)

        def mid_pair(u, carry):
            t = 2 * u
            mid_finish(t, 0)
            mid_fire(t + 2, 0, True)
            mid_finish(t + 1, 1)
            mid_fire(t + 3, 1, True)
            return carry

        lax.fori_loop(0, _cdiv(T1, 2), mid_pair, 0)
        for _ in range(2):
            pltpu.make_async_copy(fb0_0.at[pl.ds(0, CH)],
                                  out_feats.at[pl.ds(0, CH)], semw).wait()

        def cp_fire(t, S, drain):
            c = wid + t * NW

            @pl.when(c < n_cp)
            def _():
                if drain:
                    pltpu.make_async_copy(
                        fbs[S][0].at[pl.ds(0, CC)],
                        out_feats.at[pl.ds(0, CC)], semw).wait()
                pltpu.async_copy(feats_hbm.at[pl.ds(c * CC, CC)],
                                 fbs[S][0].at[pl.ds(0, CC)], semg[S])

        def cp_finish(t, S):
            c = wid + t * NW

            @pl.when(c < n_cp)
            def _():
                pltpu.make_async_copy(feats_hbm.at[pl.ds(c * CC, CC)],
                                      fbs[S][0].at[pl.ds(0, CC)],
                                      semg[S]).wait()
                pltpu.async_copy(fbs[S][0].at[pl.ds(0, CC)],
                                 out_feats.at[pl.ds(c * CC, CC)], semw)

        T2 = _cdiv(n_cp, NW)
        cp_fire(0, 0, False)
        cp_fire(1, 1, False)

        def cp_pair(u, carry):
            t = 2 * u
            cp_finish(t, 0)
            cp_fire(t + 2, 0, True)
            cp_finish(t + 1, 1)
            cp_fire(t + 3, 1, True)
            return carry

        lax.fori_loop(0, _cdiv(T2, 2), cp_pair, 0)
        for _ in range(2):
            pltpu.make_async_copy(fb0_0.at[pl.ds(0, CC)],
                                  out_feats.at[pl.ds(0, CC)], semw).wait()

        def vcp_body(i, carry):
            c = wid + i * NW

            @pl.when(c < n_vcp)
            def _():
                @pl.when(i > 0)
                def _():
                    pltpu.make_async_copy(
                        vcbuf, out_verts_tf.at[pl.ds(0, CCV)], semw).wait()

                coord = c // (V // CCV)
                j = c - coord * (V // CCV)
                pltpu.sync_copy(verts_tf.at[pl.ds(coord * V + j * CCV, CCV)],
                                vcbuf)
                pltpu.async_copy(
                    vcbuf, out_verts_tf.at[pl.ds(coord * NV + j * CCV, CCV)],
                    semw)

            return carry

        lax.fori_loop(0, _cdiv(n_vcp, NW), vcp_body, 0)
        pltpu.make_async_copy(vcbuf, out_verts_tf.at[pl.ds(0, CCV)],
                              semw).wait()

        bodies = []
        for seg in segs:
            bodies.append((seg, 0))
            bodies.append((seg, 1))
        prev_fired = None
        for (is_fte, srow, drow, reg, addnv), ii in bodies:
            src = fte_tf if is_fte else faces_tf
            j = wid + ii * NW
            if prev_fired is not None:
                @pl.when(prev_fired < n_fc)
                def _():
                    pltpu.make_async_copy(
                        ibuf, out_faces_f.at[pl.ds(0, CHF)], semw).wait()

            @pl.when(j < n_fc)
            def _(src=src, srow=srow, drow=drow, reg=reg, addnv=addnv, j=j):
                off = j * CHF
                pltpu.sync_copy(src.at[pl.ds(srow * F + off, CHF)], ibuf)
                if addnv:
                    def addb(k, carry2):
                        sl = pl.ds(k * 16, 16)
                        ibuf[sl] = ibuf[sl] + V
                        return carry2

                    lax.fori_loop(0, CHF // 16, addb, 0)
                pltpu.async_copy(
                    ibuf,
                    out_faces_f.at[pl.ds(drow * 4 * F + reg * F + off, CHF)],
                    semw)

            prev_fired = j

        @pl.when(prev_fired < n_fc)
        def _():
            pltpu.make_async_copy(ibuf, out_faces_f.at[pl.ds(0, CHF)],
                                  semw).wait()

    return kern


def kernel(verts, feats, faces, edges, faces_to_edges):
    V, D = feats.shape
    E = edges.shape[0]
    F = faces.shape[0]
    verts_tf = verts.T.reshape(-1)
    edges_f = edges.T.reshape(-1)
    faces_tf = faces.T.reshape(-1)
    fte_tf = faces_to_edges.T.reshape(-1)
    kern = _make_sc_kernel(V, D, E, F)
    out_feats, out_verts_tf, out_faces_f = kern(
        feats, verts_tf[:V], verts_tf[V:2 * V], verts_tf[2 * V:], verts_tf,
        edges_f, faces_tf, fte_tf)
    new_verts = out_verts_tf.reshape(3, V + E).T
    new_faces = out_faces_f.reshape(3, 4 * F).T
    return new_verts, new_faces, out_feats
